# Initial kernel scaffold; baseline (speedup 1.0000x reference)
import jax
import jax.numpy as jnp
from jax.experimental import pallas as pl


def kernel(x, edge_index, W1, b1, W2, b2, W3, b3):
    raise NotImplementedError("write your pallas kernel here")


# speedup vs baseline: 14.0933x; 14.0933x over previous
import functools

import jax
import jax.numpy as jnp
from jax import lax
from jax.experimental import pallas as pl
from jax.experimental.pallas import tpu as pltpu
from jax.experimental.pallas import tpu_sc as plsc

_N = 10000
_E = 320000
_D = 128
_HD = 64
_NC = 2
_NS = 16
_CH = 125
_NCH = _E // (_NC * _NS * _CH)
_NACC = 10240
_RPT = _NACC // _NS
_DEGPAD = 10240
_BLK = 1000
_GRID = _N // _BLK

_SC_PARAMS = pltpu.CompilerParams(use_tc_tiling_on_sc=False)


def _deg_body(dst_hbm, out_hbm, idx_v, ones_v, zb_v, acc_sh):
    c = lax.axis_index("c")
    s = lax.axis_index("s")
    pltpu.sync_copy(dst_hbm.at[c].at[s], idx_v)
    ones16 = jnp.ones((16,), jnp.float32)
    zeros16 = jnp.zeros((16,), jnp.float32)
    for i in range(8):
        ones_v[pl.ds(i * 16, 16)] = ones16

    def zfill(i, carry):
        zb_v[pl.ds(i * 16, 16)] = zeros16
        return carry

    lax.fori_loop(0, 40, zfill, 0)
    pltpu.sync_copy(zb_v, acc_sh.at[pl.ds(s * 640, 640)])
    plsc.subcore_barrier()

    def body(j, carry):
        pltpu.sync_copy(ones_v.at[pl.ds(0, _CH)], acc_sh.at[idx_v.at[j]],
                        add=True)
        return carry

    lax.fori_loop(0, _NCH, body, 0)
    plsc.subcore_barrier()
    pltpu.sync_copy(acc_sh.at[pl.ds(s * 640, 640)],
                    out_hbm.at[pl.ds(c * _DEGPAD + s * 640, 640)])


def _spmm_body(g_hbm, src_hbm, dst_hbm, out_hbm, srcv, dstv, rows_v, zrow_v,
               acc_sh, sem):
    c = lax.axis_index("c")
    s = lax.axis_index("s")
    pltpu.sync_copy(src_hbm.at[c].at[s], srcv)
    pltpu.sync_copy(dst_hbm.at[c].at[s], dstv)
    zeros16 = jnp.zeros((16,), jnp.float32)

    def zfill(r, carry):
        for k in range(4):
            zrow_v[r, pl.ds(k * 16, 16)] = zeros16
        return carry

    lax.fori_loop(0, 128, zfill, 0)

    for h in range(2):
        for k in range(5):
            pltpu.sync_copy(zrow_v, acc_sh.at[pl.ds(s * _RPT + k * 128, 128)])
        plsc.subcore_barrier()

        def body(j, carry):
            pltpu.async_copy(g_hbm.at[h].at[srcv.at[j]], rows_v, sem).wait()
            pltpu.sync_copy(rows_v, acc_sh.at[dstv.at[j]], add=True)
            return carry

        lax.fori_loop(0, _NCH, body, 0)
        plsc.subcore_barrier()
        pltpu.sync_copy(acc_sh.at[pl.ds(s * _RPT, _RPT)],
                        out_hbm.at[c].at[h].at[pl.ds(s * _RPT, _RPT)])


@functools.lru_cache(maxsize=None)
def _deg_call():
    mesh = plsc.VectorSubcoreMesh(core_axis_name="c", subcore_axis_name="s")
    return pl.kernel(
        _deg_body,
        out_type=jax.ShapeDtypeStruct((_NC * _DEGPAD,), jnp.float32),
        mesh=mesh,
        compiler_params=_SC_PARAMS,
        scratch_types=[
            pltpu.VMEM((_NCH, _CH), jnp.int32),
            pltpu.VMEM((128,), jnp.float32),
            pltpu.VMEM((640,), jnp.float32),
            pltpu.VMEM_SHARED((_DEGPAD,), jnp.float32),
        ],
    )


@functools.lru_cache(maxsize=None)
def _spmm_call():
    mesh = plsc.VectorSubcoreMesh(core_axis_name="c", subcore_axis_name="s")
    return pl.kernel(
        _spmm_body,
        out_type=jax.ShapeDtypeStruct((_NC, 2, _NACC, _HD), jnp.float32),
        mesh=mesh,
        compiler_params=_SC_PARAMS,
        scratch_types=[
            pltpu.VMEM((_NCH, _CH), jnp.int32),
            pltpu.VMEM((_NCH, _CH), jnp.int32),
            pltpu.VMEM((_CH, _HD), jnp.float32),
            pltpu.VMEM((128, _HD), jnp.float32),
            pltpu.VMEM_SHARED((_NACC, _HD), jnp.float32),
            pltpu.SemaphoreType.DMA,
        ],
    )


def _tc1_body(x_ref, w_ref, deg_ref, g_ref):
    dinv = lax.rsqrt(deg_ref[...])
    res = dinv * jnp.dot(x_ref[...], w_ref[...],
                         preferred_element_type=jnp.float32)
    g_ref[0, :, :] = res[:, :_HD]
    g_ref[1, :, :] = res[:, _HD:]


def _agg(p_ref, g_ref):
    left = p_ref[0, 0] + p_ref[1, 0] + g_ref[0]
    right = p_ref[0, 1] + p_ref[1, 1] + g_ref[1]
    return jnp.concatenate([left, right], axis=1)


def _tc_mid_body(p_ref, g_ref, deg_ref, b_ref, w_ref, gout_ref):
    dinv = lax.rsqrt(deg_ref[...])
    y = dinv * _agg(p_ref, g_ref) + b_ref[...]
    res = dinv * jnp.dot(y, w_ref[...], preferred_element_type=jnp.float32)
    gout_ref[0, :, :] = res[:, :_HD]
    gout_ref[1, :, :] = res[:, _HD:]


def _tc_out_body(p_ref, g_ref, deg_ref, b_ref, out_ref):
    dinv = lax.rsqrt(deg_ref[...])
    out_ref[...] = dinv * _agg(p_ref, g_ref) + b_ref[...]


def _gspec():
    return pl.BlockSpec((2, _BLK, _HD), lambda i: (0, i, 0))


def _pspec():
    return pl.BlockSpec((_NC, 2, _BLK, _HD), lambda i: (0, 0, i, 0))


def _tc1(x, w, deg):
    return pl.pallas_call(
        _tc1_body,
        grid=(_GRID,),
        in_specs=[
            pl.BlockSpec((_BLK, _D), lambda i: (i, 0)),
            pl.BlockSpec((_D, _D), lambda i: (0, 0)),
            pl.BlockSpec((_BLK, 1), lambda i: (i, 0)),
        ],
        out_specs=_gspec(),
        out_shape=jax.ShapeDtypeStruct((2, _N, _HD), jnp.float32),
    )(x, w, deg)


def _tc_mid(p, g, deg, b, w):
    return pl.pallas_call(
        _tc_mid_body,
        grid=(_GRID,),
        in_specs=[
            _pspec(),
            _gspec(),
            pl.BlockSpec((_BLK, 1), lambda i: (i, 0)),
            pl.BlockSpec((1, _D), lambda i: (0, 0)),
            pl.BlockSpec((_D, _D), lambda i: (0, 0)),
        ],
        out_specs=_gspec(),
        out_shape=jax.ShapeDtypeStruct((2, _N, _HD), jnp.float32),
    )(p, g, deg, b, w)


def _tc_out(p, g, deg, b):
    return pl.pallas_call(
        _tc_out_body,
        grid=(_GRID,),
        in_specs=[
            _pspec(),
            _gspec(),
            pl.BlockSpec((_BLK, 1), lambda i: (i, 0)),
            pl.BlockSpec((1, _D), lambda i: (0, 0)),
        ],
        out_specs=pl.BlockSpec((_BLK, _D), lambda i: (i, 0)),
        out_shape=jax.ShapeDtypeStruct((_N, _D), jnp.float32),
    )(p, g, deg, b)


def kernel(x, edge_index, W1, b1, W2, b2, W3, b3):
    src = edge_index[0].reshape(_NC, _NS, _NCH, _CH)
    dst = edge_index[1].reshape(_NC, _NS, _NCH, _CH)

    degp = _deg_call()(dst)
    deg = (degp[:_N] + degp[_DEGPAD:_DEGPAD + _N] + 1.0).reshape(_N, 1)

    b1r = b1.reshape(1, _D)
    b2r = b2.reshape(1, _D)
    b3r = b3.reshape(1, _D)

    g = _tc1(x, W1, deg)
    p = _spmm_call()(g, src, dst)
    g = _tc_mid(p, g, deg, b1r, W2)
    p = _spmm_call()(g, src, dst)
    g = _tc_mid(p, g, deg, b2r, W3)
    p = _spmm_call()(g, src, dst)
    return _tc_out(p, g, deg, b3r)


# speedup vs baseline: 21.2542x; 1.5081x over previous
import functools

import jax
import jax.numpy as jnp
from jax import lax
from jax.experimental import pallas as pl
from jax.experimental.pallas import tpu as pltpu
from jax.experimental.pallas import tpu_sc as plsc

_N = 10000
_E = 320000
_D = 128
_HD = 64
_NC = 2
_NS = 16
_CH = 125
_NCH = _E // (_NC * _NS * _CH)
_NACC = 10240
_RPT = _NACC // _NS
_DEGPAD = 10240
_BLK = 1000
_GRID = _N // _BLK

_SC_PARAMS = pltpu.CompilerParams(use_tc_tiling_on_sc=False)


def _deg_body(dst_hbm, out_hbm, idx_v, ones_v, zb_v, acc_sh):
    c = lax.axis_index("c")
    s = lax.axis_index("s")
    pltpu.sync_copy(dst_hbm.at[c].at[s], idx_v)
    ones16 = jnp.ones((16,), jnp.float32)
    zeros16 = jnp.zeros((16,), jnp.float32)
    for i in range(8):
        ones_v[pl.ds(i * 16, 16)] = ones16

    def zfill(i, carry):
        zb_v[pl.ds(i * 16, 16)] = zeros16
        return carry

    lax.fori_loop(0, 40, zfill, 0)
    pltpu.sync_copy(zb_v, acc_sh.at[pl.ds(s * 640, 640)])
    plsc.subcore_barrier()

    def body(j, carry):
        pltpu.sync_copy(ones_v.at[pl.ds(0, _CH)], acc_sh.at[idx_v.at[j]],
                        add=True)
        return carry

    lax.fori_loop(0, _NCH, body, 0)
    plsc.subcore_barrier()
    pltpu.sync_copy(acc_sh.at[pl.ds(s * 640, 640)],
                    out_hbm.at[pl.ds(c * _DEGPAD + s * 640, 640)])


def _spmm_body(g_hbm, src_hbm, dst_hbm, out_hbm, srcv, dstv, rows_a, rows_b,
               zrow_v, acc_sh, sem_a, sem_b):
    c = lax.axis_index("c")
    s = lax.axis_index("s")
    pltpu.sync_copy(src_hbm.at[c].at[s], srcv)
    pltpu.sync_copy(dst_hbm.at[c].at[s], dstv)
    zeros16 = jnp.zeros((16,), jnp.float32)

    def zfill(r, carry):
        for k in range(4):
            zrow_v[r, pl.ds(k * 16, 16)] = zeros16
        return carry

    lax.fori_loop(0, 128, zfill, 0)

    for h in range(2):
        for k in range(5):
            pltpu.sync_copy(zrow_v, acc_sh.at[pl.ds(s * _RPT + k * 128, 128)])
        plsc.subcore_barrier()

        pltpu.async_copy(g_hbm.at[h].at[srcv.at[0]], rows_a, sem_a)
        pltpu.async_copy(g_hbm.at[h].at[srcv.at[1]], rows_b, sem_b)

        def body(jj, carry):
            j = jj * 2
            pltpu.make_async_copy(g_hbm.at[h].at[srcv.at[j]],
                                  rows_a, sem_a).wait()
            pltpu.sync_copy(rows_a, acc_sh.at[dstv.at[j]], add=True)

            @pl.when(j + 2 < _NCH)
            def _():
                pltpu.async_copy(g_hbm.at[h].at[srcv.at[j + 2]],
                                 rows_a, sem_a)

            pltpu.make_async_copy(g_hbm.at[h].at[srcv.at[j + 1]],
                                  rows_b, sem_b).wait()
            pltpu.sync_copy(rows_b, acc_sh.at[dstv.at[j + 1]], add=True)

            @pl.when(j + 3 < _NCH)
            def _():
                pltpu.async_copy(g_hbm.at[h].at[srcv.at[j + 3]],
                                 rows_b, sem_b)

            return carry

        lax.fori_loop(0, _NCH // 2, body, 0)
        plsc.subcore_barrier()
        pltpu.sync_copy(acc_sh.at[pl.ds(s * _RPT, _RPT)],
                        out_hbm.at[c].at[h].at[pl.ds(s * _RPT, _RPT)])


@functools.lru_cache(maxsize=None)
def _deg_call():
    mesh = plsc.VectorSubcoreMesh(core_axis_name="c", subcore_axis_name="s")
    return pl.kernel(
        _deg_body,
        out_type=jax.ShapeDtypeStruct((_NC * _DEGPAD,), jnp.float32),
        mesh=mesh,
        compiler_params=_SC_PARAMS,
        scratch_types=[
            pltpu.VMEM((_NCH, _CH), jnp.int32),
            pltpu.VMEM((128,), jnp.float32),
            pltpu.VMEM((640,), jnp.float32),
            pltpu.VMEM_SHARED((_DEGPAD,), jnp.float32),
        ],
    )


@functools.lru_cache(maxsize=None)
def _spmm_call():
    mesh = plsc.VectorSubcoreMesh(core_axis_name="c", subcore_axis_name="s")
    return pl.kernel(
        _spmm_body,
        out_type=jax.ShapeDtypeStruct((_NC, 2, _NACC, _HD), jnp.float32),
        mesh=mesh,
        compiler_params=_SC_PARAMS,
        scratch_types=[
            pltpu.VMEM((_NCH, _CH), jnp.int32),
            pltpu.VMEM((_NCH, _CH), jnp.int32),
            pltpu.VMEM((_CH, _HD), jnp.float32),
            pltpu.VMEM((_CH, _HD), jnp.float32),
            pltpu.VMEM((128, _HD), jnp.float32),
            pltpu.VMEM_SHARED((_NACC, _HD), jnp.float32),
            pltpu.SemaphoreType.DMA,
            pltpu.SemaphoreType.DMA,
        ],
    )


def _tc1_body(x_ref, w_ref, deg_ref, g_ref):
    dinv = lax.rsqrt(deg_ref[...])
    res = dinv * jnp.dot(x_ref[...], w_ref[...],
                         preferred_element_type=jnp.float32)
    g_ref[0, :, :] = res[:, :_HD]
    g_ref[1, :, :] = res[:, _HD:]


def _agg(p_ref, g_ref):
    left = p_ref[0, 0] + p_ref[1, 0] + g_ref[0]
    right = p_ref[0, 1] + p_ref[1, 1] + g_ref[1]
    return jnp.concatenate([left, right], axis=1)


def _tc_mid_body(p_ref, g_ref, deg_ref, b_ref, w_ref, gout_ref):
    dinv = lax.rsqrt(deg_ref[...])
    y = dinv * _agg(p_ref, g_ref) + b_ref[...]
    res = dinv * jnp.dot(y, w_ref[...], preferred_element_type=jnp.float32)
    gout_ref[0, :, :] = res[:, :_HD]
    gout_ref[1, :, :] = res[:, _HD:]


def _tc_out_body(p_ref, g_ref, deg_ref, b_ref, out_ref):
    dinv = lax.rsqrt(deg_ref[...])
    out_ref[...] = dinv * _agg(p_ref, g_ref) + b_ref[...]


def _gspec():
    return pl.BlockSpec((2, _BLK, _HD), lambda i: (0, i, 0))


def _pspec():
    return pl.BlockSpec((_NC, 2, _BLK, _HD), lambda i: (0, 0, i, 0))


def _tc1(x, w, deg):
    return pl.pallas_call(
        _tc1_body,
        grid=(_GRID,),
        in_specs=[
            pl.BlockSpec((_BLK, _D), lambda i: (i, 0)),
            pl.BlockSpec((_D, _D), lambda i: (0, 0)),
            pl.BlockSpec((_BLK, 1), lambda i: (i, 0)),
        ],
        out_specs=_gspec(),
        out_shape=jax.ShapeDtypeStruct((2, _N, _HD), jnp.float32),
    )(x, w, deg)


def _tc_mid(p, g, deg, b, w):
    return pl.pallas_call(
        _tc_mid_body,
        grid=(_GRID,),
        in_specs=[
            _pspec(),
            _gspec(),
            pl.BlockSpec((_BLK, 1), lambda i: (i, 0)),
            pl.BlockSpec((1, _D), lambda i: (0, 0)),
            pl.BlockSpec((_D, _D), lambda i: (0, 0)),
        ],
        out_specs=_gspec(),
        out_shape=jax.ShapeDtypeStruct((2, _N, _HD), jnp.float32),
    )(p, g, deg, b, w)


def _tc_out(p, g, deg, b):
    return pl.pallas_call(
        _tc_out_body,
        grid=(_GRID,),
        in_specs=[
            _pspec(),
            _gspec(),
            pl.BlockSpec((_BLK, 1), lambda i: (i, 0)),
            pl.BlockSpec((1, _D), lambda i: (0, 0)),
        ],
        out_specs=pl.BlockSpec((_BLK, _D), lambda i: (i, 0)),
        out_shape=jax.ShapeDtypeStruct((_N, _D), jnp.float32),
    )(p, g, deg, b)


def kernel(x, edge_index, W1, b1, W2, b2, W3, b3):
    src = edge_index[0].reshape(_NC, _NS, _NCH, _CH)
    dst = edge_index[1].reshape(_NC, _NS, _NCH, _CH)

    degp = _deg_call()(dst)
    deg = (degp[:_N] + degp[_DEGPAD:_DEGPAD + _N] + 1.0).reshape(_N, 1)

    b1r = b1.reshape(1, _D)
    b2r = b2.reshape(1, _D)
    b3r = b3.reshape(1, _D)

    g = _tc1(x, W1, deg)
    p = _spmm_call()(g, src, dst)
    g = _tc_mid(p, g, deg, b1r, W2)
    p = _spmm_call()(g, src, dst)
    g = _tc_mid(p, g, deg, b2r, W3)
    p = _spmm_call()(g, src, dst)
    return _tc_out(p, g, deg, b3r)


# speedup vs baseline: 25.3907x; 1.1946x over previous
import functools

import jax
import jax.numpy as jnp
from jax import lax
from jax.experimental import pallas as pl
from jax.experimental.pallas import tpu as pltpu
from jax.experimental.pallas import tpu_sc as plsc

_N = 10000
_E = 320000
_D = 128
_HD = 64
_NC = 2
_NS = 16
_CH = 125
_NCH = _E // (_NC * _NS * _CH)
_NACC = 10240
_RPT = _NACC // _NS
_DEGPAD = 10240
_BLK = 1000
_GRID = _N // _BLK

_SC_PARAMS = pltpu.CompilerParams(use_tc_tiling_on_sc=False)


def _deg_body(dst_hbm, out_hbm, idx_v, ones_v, zb_v, acc_sh):
    c = lax.axis_index("c")
    s = lax.axis_index("s")
    pltpu.sync_copy(dst_hbm.at[c].at[s], idx_v)
    ones16 = jnp.ones((16,), jnp.float32)
    zeros16 = jnp.zeros((16,), jnp.float32)
    for i in range(8):
        ones_v[pl.ds(i * 16, 16)] = ones16

    def zfill(i, carry):
        zb_v[pl.ds(i * 16, 16)] = zeros16
        return carry

    lax.fori_loop(0, 40, zfill, 0)
    pltpu.sync_copy(zb_v, acc_sh.at[pl.ds(s * 640, 640)])
    plsc.subcore_barrier()

    def body(j, carry):
        pltpu.sync_copy(ones_v.at[pl.ds(0, _CH)], acc_sh.at[idx_v.at[j]],
                        add=True)
        return carry

    lax.fori_loop(0, _NCH, body, 0)
    plsc.subcore_barrier()
    pltpu.sync_copy(acc_sh.at[pl.ds(s * 640, 640)],
                    out_hbm.at[pl.ds(c * _DEGPAD + s * 640, 640)])


_DEPTH = 5


def _spmm_body(g_hbm, src_hbm, dst_hbm, out_hbm, srcv, dstv, *rest):
    bufs = rest[:_DEPTH]
    zrow_v = rest[_DEPTH]
    acc_sh = rest[_DEPTH + 1]
    sems = rest[_DEPTH + 2:]
    c = lax.axis_index("c")
    s = lax.axis_index("s")
    pltpu.sync_copy(src_hbm.at[c].at[s], srcv)
    pltpu.sync_copy(dst_hbm.at[c].at[s], dstv)
    zeros16 = jnp.zeros((16,), jnp.float32)

    def zfill(r, carry):
        for k in range(4):
            zrow_v[r, pl.ds(k * 16, 16)] = zeros16
        return carry

    lax.fori_loop(0, 128, zfill, 0)

    for h in range(2):
        for k in range(5):
            pltpu.sync_copy(zrow_v, acc_sh.at[pl.ds(s * _RPT + k * 128, 128)])
        plsc.subcore_barrier()

        for u in range(_DEPTH):
            pltpu.async_copy(g_hbm.at[h].at[srcv.at[u]], bufs[u], sems[u])

        def body(jj, carry):
            j = jj * _DEPTH
            for u in range(_DEPTH):
                pltpu.make_async_copy(g_hbm.at[h].at[srcv.at[j + u]],
                                      bufs[u], sems[u]).wait()
                pltpu.sync_copy(bufs[u], acc_sh.at[dstv.at[j + u]], add=True)

                @pl.when(j + u + _DEPTH < _NCH)
                def _():
                    pltpu.async_copy(g_hbm.at[h].at[srcv.at[j + u + _DEPTH]],
                                     bufs[u], sems[u])

            return carry

        lax.fori_loop(0, _NCH // _DEPTH, body, 0)
        plsc.subcore_barrier()
        pltpu.sync_copy(acc_sh.at[pl.ds(s * _RPT, _RPT)],
                        out_hbm.at[c].at[h].at[pl.ds(s * _RPT, _RPT)])


@functools.lru_cache(maxsize=None)
def _deg_call():
    mesh = plsc.VectorSubcoreMesh(core_axis_name="c", subcore_axis_name="s")
    return pl.kernel(
        _deg_body,
        out_type=jax.ShapeDtypeStruct((_NC * _DEGPAD,), jnp.float32),
        mesh=mesh,
        compiler_params=_SC_PARAMS,
        scratch_types=[
            pltpu.VMEM((_NCH, _CH), jnp.int32),
            pltpu.VMEM((128,), jnp.float32),
            pltpu.VMEM((640,), jnp.float32),
            pltpu.VMEM_SHARED((_DEGPAD,), jnp.float32),
        ],
    )


@functools.lru_cache(maxsize=None)
def _spmm_call():
    mesh = plsc.VectorSubcoreMesh(core_axis_name="c", subcore_axis_name="s")
    return pl.kernel(
        _spmm_body,
        out_type=jax.ShapeDtypeStruct((_NC, 2, _NACC, _HD), jnp.float32),
        mesh=mesh,
        compiler_params=_SC_PARAMS,
        scratch_types=[
            pltpu.VMEM((_NCH, _CH), jnp.int32),
            pltpu.VMEM((_NCH, _CH), jnp.int32),
        ] + [pltpu.VMEM((_CH, _HD), jnp.float32) for _ in range(_DEPTH)] + [
            pltpu.VMEM((128, _HD), jnp.float32),
            pltpu.VMEM_SHARED((_NACC, _HD), jnp.float32),
        ] + [pltpu.SemaphoreType.DMA for _ in range(_DEPTH)],
    )


def _tc1_body(x_ref, w_ref, deg_ref, g_ref):
    dinv = lax.rsqrt(deg_ref[...])
    res = dinv * jnp.dot(x_ref[...], w_ref[...],
                         preferred_element_type=jnp.float32)
    g_ref[0, :, :] = res[:, :_HD]
    g_ref[1, :, :] = res[:, _HD:]


def _agg(p_ref, g_ref):
    left = p_ref[0, 0] + p_ref[1, 0] + g_ref[0]
    right = p_ref[0, 1] + p_ref[1, 1] + g_ref[1]
    return jnp.concatenate([left, right], axis=1)


def _tc_mid_body(p_ref, g_ref, deg_ref, b_ref, w_ref, gout_ref):
    dinv = lax.rsqrt(deg_ref[...])
    y = dinv * _agg(p_ref, g_ref) + b_ref[...]
    res = dinv * jnp.dot(y, w_ref[...], preferred_element_type=jnp.float32)
    gout_ref[0, :, :] = res[:, :_HD]
    gout_ref[1, :, :] = res[:, _HD:]


def _tc_out_body(p_ref, g_ref, deg_ref, b_ref, out_ref):
    dinv = lax.rsqrt(deg_ref[...])
    out_ref[...] = dinv * _agg(p_ref, g_ref) + b_ref[...]


def _gspec():
    return pl.BlockSpec((2, _BLK, _HD), lambda i: (0, i, 0))


def _pspec():
    return pl.BlockSpec((_NC, 2, _BLK, _HD), lambda i: (0, 0, i, 0))


def _tc1(x, w, deg):
    return pl.pallas_call(
        _tc1_body,
        grid=(_GRID,),
        in_specs=[
            pl.BlockSpec((_BLK, _D), lambda i: (i, 0)),
            pl.BlockSpec((_D, _D), lambda i: (0, 0)),
            pl.BlockSpec((_BLK, 1), lambda i: (i, 0)),
        ],
        out_specs=_gspec(),
        out_shape=jax.ShapeDtypeStruct((2, _N, _HD), jnp.float32),
    )(x, w, deg)


def _tc_mid(p, g, deg, b, w):
    return pl.pallas_call(
        _tc_mid_body,
        grid=(_GRID,),
        in_specs=[
            _pspec(),
            _gspec(),
            pl.BlockSpec((_BLK, 1), lambda i: (i, 0)),
            pl.BlockSpec((1, _D), lambda i: (0, 0)),
            pl.BlockSpec((_D, _D), lambda i: (0, 0)),
        ],
        out_specs=_gspec(),
        out_shape=jax.ShapeDtypeStruct((2, _N, _HD), jnp.float32),
    )(p, g, deg, b, w)


def _tc_out(p, g, deg, b):
    return pl.pallas_call(
        _tc_out_body,
        grid=(_GRID,),
        in_specs=[
            _pspec(),
            _gspec(),
            pl.BlockSpec((_BLK, 1), lambda i: (i, 0)),
            pl.BlockSpec((1, _D), lambda i: (0, 0)),
        ],
        out_specs=pl.BlockSpec((_BLK, _D), lambda i: (i, 0)),
        out_shape=jax.ShapeDtypeStruct((_N, _D), jnp.float32),
    )(p, g, deg, b)


def kernel(x, edge_index, W1, b1, W2, b2, W3, b3):
    src = edge_index[0].reshape(_NC, _NS, _NCH, _CH)
    dst = edge_index[1].reshape(_NC, _NS, _NCH, _CH)

    degp = _deg_call()(dst)
    deg = (degp[:_N] + degp[_DEGPAD:_DEGPAD + _N] + 1.0).reshape(_N, 1)

    b1r = b1.reshape(1, _D)
    b2r = b2.reshape(1, _D)
    b3r = b3.reshape(1, _D)

    g = _tc1(x, W1, deg)
    p = _spmm_call()(g, src, dst)
    g = _tc_mid(p, g, deg, b1r, W2)
    p = _spmm_call()(g, src, dst)
    g = _tc_mid(p, g, deg, b2r, W3)
    p = _spmm_call()(g, src, dst)
    return _tc_out(p, g, deg, b3r)
